# baseline (speedup 1.0000x reference)
import jax
import jax.numpy as jnp
from jax.experimental import pallas as pl
from jax.experimental.pallas import tpu as pltpu

_TAPS = [(i, j) for i in range(3) for j in range(3)]


def _net_kernel(x_ref, w1_ref, b1_ref, w2_ref, b2_ref, out_ref, z_ref):
    x = x_ref[0]
    y = jnp.zeros((62, 62, 64), jnp.float32)
    for t, (di, dj) in enumerate(_TAPS):
        y = y + x[di:di + 62, dj:dj + 62][:, :, None] * w1_ref[t][None, None, :]
    y = y + b1_ref[:]

    p = jnp.dot(y.reshape(62 * 62, 64), w2_ref[:],
                preferred_element_type=jnp.float32)
    p3 = p.reshape(62, 62, 288)
    z_ref[...] = jnp.zeros((64, 64, 32), jnp.float32)
    for t, (ei, ej) in enumerate(_TAPS):
        a, b = 2 - ei, 2 - ej
        z_ref[a:a + 62, b:b + 62, :] += p3[:, :, 32 * t:32 * (t + 1)]
    z2 = z_ref[...].reshape(64 * 64, 32) + b2_ref[:]
    out_ref[0] = z2.T


def kernel(x, W1, b1, W2, b2):
    n = x.shape[0]
    x2 = x.reshape(n, 64, 64)
    w1 = W1.reshape(9, 64)
    w2 = jnp.transpose(W2.reshape(9, 64, 32), (1, 0, 2)).reshape(64, 288)
    out = pl.pallas_call(
        _net_kernel,
        grid=(n,),
        in_specs=[
            pl.BlockSpec((1, 64, 64), lambda i: (i, 0, 0)),
            pl.BlockSpec((9, 64), lambda i: (0, 0)),
            pl.BlockSpec((1, 64), lambda i: (0, 0)),
            pl.BlockSpec((64, 288), lambda i: (0, 0)),
            pl.BlockSpec((1, 32), lambda i: (0, 0)),
        ],
        out_specs=pl.BlockSpec((1, 32, 4096), lambda i: (i, 0, 0)),
        out_shape=jax.ShapeDtypeStruct((n, 32, 4096), jnp.float32),
        scratch_shapes=[pltpu.VMEM((64, 64, 32), jnp.float32)],
    )(x2, w1, b1.reshape(1, 64), w2, b2.reshape(1, 32))
    return out.reshape(n, 32, 64, 64)


# speedup vs baseline: 2.9497x; 2.9497x over previous
import jax
import jax.numpy as jnp
from jax.experimental import pallas as pl

_TAPS = [(i, j) for i in range(3) for j in range(3)]
_NB = 8


def _net_kernel(xp_ref, wct_ref, bias_ref, out_ref):
    for b in range(_NB):
        p = jnp.dot(wct_ref[:], xp_ref[b],
                    preferred_element_type=jnp.float32)
        z = bias_ref[:]
        for t, (ei, ej) in enumerate(_TAPS):
            s = 64 * ei + ej
            z = z + p[32 * t:32 * t + 32, s:s + 4096]
        out_ref[b] = z


def kernel(x, W1, b1, W2, b2):
    n = x.shape[0]
    x2 = x.reshape(n, 64, 64)
    cols = []
    for di in range(3):
        for dj in range(3):
            sl = x2[:, di:di + 62, dj:dj + 62]
            sl = jnp.pad(sl, ((0, 0), (0, 0), (0, 2)))
            cols.append(sl.reshape(n, 62 * 64))
    xp = jnp.stack(cols, axis=1)
    xp = jnp.pad(xp, ((0, 0), (0, 0), (130, 254)))

    w2r = W2.reshape(9, 64, 32)
    wct = jnp.einsum('uc,tco->tou', W1.reshape(9, 64), w2r).reshape(288, 9)

    bt = jnp.einsum('c,tco->to', b1, w2r)
    pq = jnp.arange(64)
    plane = jnp.zeros((32, 64, 64), jnp.float32) + b2[:, None, None]
    for t, (ei, ej) in enumerate(_TAPS):
        rowok = (pq + ei - 2 >= 0) & (pq + ei - 2 <= 61)
        colok = (pq + ej - 2 >= 0) & (pq + ej - 2 <= 61)
        mask = (rowok[:, None] & colok[None, :]).astype(jnp.float32)
        plane = plane + bt[t][:, None, None] * mask[None, :, :]
    bias = plane.reshape(32, 4096)

    out = pl.pallas_call(
        _net_kernel,
        grid=(n // _NB,),
        in_specs=[
            pl.BlockSpec((_NB, 9, 4352), lambda i: (i, 0, 0)),
            pl.BlockSpec((288, 9), lambda i: (0, 0)),
            pl.BlockSpec((32, 4096), lambda i: (0, 0)),
        ],
        out_specs=pl.BlockSpec((_NB, 32, 4096), lambda i: (i, 0, 0)),
        out_shape=jax.ShapeDtypeStruct((n, 32, 4096), jnp.float32),
    )(xp, wct, bias)
    return out.reshape(n, 32, 64, 64)


# speedup vs baseline: 3.0935x; 1.0487x over previous
import jax
import jax.numpy as jnp
from jax.experimental import pallas as pl
from jax.experimental.pallas import tpu as pltpu

_TAPS = [(i, j) for i in range(3) for j in range(3)]
_NB = 8


def _net_kernel(xin_ref, wct_ref, mask_ref, bias_ref, out_ref, rhs_ref):
    for b in range(_NB):
        for t, (di, dj) in enumerate(_TAPS):
            s = 64 * di + dj
            rhs_ref[t, :] = xin_ref[b, s:s + 4352] * mask_ref[0]
        p = jnp.dot(wct_ref[:], rhs_ref[...],
                    preferred_element_type=jnp.float32)
        z = bias_ref[:]
        for t, (ei, ej) in enumerate(_TAPS):
            s = 64 * ei + ej
            z = z + p[32 * t:32 * t + 32, s:s + 4096]
        out_ref[b] = z


def kernel(x, W1, b1, W2, b2):
    n = x.shape[0]
    xin = jnp.pad(x.reshape(n, 4096), ((0, 0), (130, 382)))

    w2r = W2.reshape(9, 64, 32)
    wct = jnp.einsum('uc,tco->tou', W1.reshape(9, 64), w2r).reshape(288, 9)

    ll = jnp.arange(4352)
    mask = ((ll >= 130) & (ll < 4098) & ((ll - 130) % 64 < 62))
    mask = mask.astype(jnp.float32)[None, :]

    bt = jnp.einsum('c,tco->to', b1, w2r)
    pq = jnp.arange(64)
    plane = jnp.zeros((32, 64, 64), jnp.float32) + b2[:, None, None]
    for t, (ei, ej) in enumerate(_TAPS):
        rowok = (pq + ei - 2 >= 0) & (pq + ei - 2 <= 61)
        colok = (pq + ej - 2 >= 0) & (pq + ej - 2 <= 61)
        m = (rowok[:, None] & colok[None, :]).astype(jnp.float32)
        plane = plane + bt[t][:, None, None] * m[None, :, :]
    bias = plane.reshape(32, 4096)

    out = pl.pallas_call(
        _net_kernel,
        grid=(n // _NB,),
        in_specs=[
            pl.BlockSpec((_NB, 4608), lambda i: (i, 0)),
            pl.BlockSpec((288, 9), lambda i: (0, 0)),
            pl.BlockSpec((1, 4352), lambda i: (0, 0)),
            pl.BlockSpec((32, 4096), lambda i: (0, 0)),
        ],
        out_specs=pl.BlockSpec((_NB, 32, 4096), lambda i: (i, 0, 0)),
        out_shape=jax.ShapeDtypeStruct((n, 32, 4096), jnp.float32),
        scratch_shapes=[pltpu.VMEM((9, 4352), jnp.float32)],
    )(xin, wct, mask, bias)
    return out.reshape(n, 32, 64, 64)


# speedup vs baseline: 3.5587x; 1.1504x over previous
import jax
import jax.numpy as jnp
from jax.experimental import pallas as pl
from jax.experimental.pallas import tpu as pltpu

_TAPS = [(i, j) for i in range(3) for j in range(3)]
_NB = 8


def _net_kernel(xin_ref, wct_ref, mask_ref, bias_ref, out_ref, rhs_ref):
    for b in range(_NB):
        for t, (di, dj) in enumerate(_TAPS):
            s = 64 * di + dj
            rhs_ref[t, b * 4352:b * 4352 + 4352] = (
                xin_ref[b, s:s + 4352] * mask_ref[0])
    for c in range(_NB // 2):
        p = jnp.dot(wct_ref[:], rhs_ref[:, c * 8704:(c + 1) * 8704],
                    preferred_element_type=jnp.float32)
        for h in range(2):
            b = 2 * c + h
            off = h * 4352
            slabs = [p[32 * t:32 * t + 32,
                       off + 64 * ei + ej:off + 64 * ei + ej + 4096]
                     for t, (ei, ej) in enumerate(_TAPS)]
            z = ((bias_ref[:] + slabs[0]) + (slabs[1] + slabs[2])) \
                + ((slabs[3] + slabs[4]) + (slabs[5] + slabs[6])) \
                + (slabs[7] + slabs[8])
            out_ref[b] = z


def kernel(x, W1, b1, W2, b2):
    n = x.shape[0]
    xin = jnp.pad(x.reshape(n, 4096), ((0, 0), (130, 382))).astype(jnp.bfloat16)

    w2r = W2.reshape(9, 64, 32)
    wct = jnp.einsum("uc,tco->tou", W1.reshape(9, 64), w2r).reshape(288, 9).astype(jnp.bfloat16)

    ll = jnp.arange(4352)
    mask = ((ll >= 130) & (ll < 4098) & ((ll - 130) % 64 < 62))
    mask = mask.astype(jnp.bfloat16)[None, :]

    bt = jnp.einsum('c,tco->to', b1, w2r)
    pq = jnp.arange(64)
    plane = jnp.zeros((32, 64, 64), jnp.float32) + b2[:, None, None]
    for t, (ei, ej) in enumerate(_TAPS):
        rowok = (pq + ei - 2 >= 0) & (pq + ei - 2 <= 61)
        colok = (pq + ej - 2 >= 0) & (pq + ej - 2 <= 61)
        m = (rowok[:, None] & colok[None, :]).astype(jnp.float32)
        plane = plane + bt[t][:, None, None] * m[None, :, :]
    bias = plane.reshape(32, 4096)

    out = pl.pallas_call(
        _net_kernel,
        grid=(n // _NB,),
        in_specs=[
            pl.BlockSpec((_NB, 4608), lambda i: (i, 0)),
            pl.BlockSpec((288, 9), lambda i: (0, 0)),
            pl.BlockSpec((1, 4352), lambda i: (0, 0)),
            pl.BlockSpec((32, 4096), lambda i: (0, 0)),
        ],
        out_specs=pl.BlockSpec((_NB, 32, 4096), lambda i: (i, 0, 0)),
        out_shape=jax.ShapeDtypeStruct((n, 32, 4096), jnp.float32),
        scratch_shapes=[pltpu.VMEM((9, _NB * 4352), jnp.bfloat16)],
    )(xin, wct, mask, bias)
    return out.reshape(n, 32, 64, 64)
